# Initial kernel scaffold; baseline (speedup 1.0000x reference)
import jax
import jax.numpy as jnp
from jax.experimental import pallas as pl


def kernel(feat, edge_index, W1, b1, W2, b2):
    raise NotImplementedError("write your pallas kernel here")


# speedup vs baseline: 7.7325x; 7.7325x over previous
import functools

import jax
import jax.numpy as jnp
from jax import lax
from jax.experimental import pallas as pl
from jax.experimental.pallas import tpu as pltpu
from jax.experimental.pallas import tpu_sc as plsc

N = 10000
E = 320000
D = 128

NPAD = 10240
EPAD = 327680
NC, NS = 2, 16
NW = NC * NS
EPT = EPAD // NW
CHUNK = 128
NCHUNK = EPT // CHUNK
RPT = NPAD // NS
CW = 8

def _mesh():
    return plsc.VectorSubcoreMesh(
        core_axis_name="c", subcore_axis_name="s", num_cores=NC, num_subcores=NS)


@functools.cache
def _sc_count_kernel():
    @functools.partial(
        pl.kernel,
        out_type=jax.ShapeDtypeStruct((NW, NPAD), jnp.float32),
        mesh=_mesh(),
        compiler_params=pltpu.CompilerParams(needs_layout_passes=False),
        scratch_types=[
            pltpu.VMEM((NCHUNK, CHUNK), jnp.int32),
            pltpu.VMEM((NPAD,), jnp.float32),
        ],
    )
    def _sc_count(dst_hbm, zeros_hbm, out_hbm, idx_v, cnt_v):
        c = lax.axis_index("c")
        s = lax.axis_index("s")
        w = c * NS + s
        pltpu.sync_copy(zeros_hbm, cnt_v)
        pltpu.sync_copy(dst_hbm.at[w], idx_v)
        ones = jnp.full((16,), 1.0, jnp.float32)

        def body(j, carry):
            for k in range(CHUNK // 16):
                idx = idx_v[j, pl.ds(k * 16, 16)]
                plsc.addupdate_scatter(cnt_v, [idx], ones)
            return carry

        lax.fori_loop(0, NCHUNK, body, 0)
        pltpu.sync_copy(cnt_v, out_hbm.at[w])

    return _sc_count


def _sc_count(dst, zeros_n):
    return _sc_count_kernel()(dst, zeros_n)


@functools.cache
def _sc_agg_kernel():
    @functools.partial(
        pl.kernel,
        out_type=jax.ShapeDtypeStruct((NC, NPAD, D), jnp.float32),
        mesh=_mesh(),
        scratch_types=[
            pltpu.VMEM((NCHUNK, CHUNK), jnp.int32),
            pltpu.VMEM((NCHUNK, CHUNK), jnp.int32),
            pltpu.VMEM((CHUNK, D), jnp.float32),
            pltpu.VMEM_SHARED((NPAD, D), jnp.float32),
            pltpu.SemaphoreType.DMA,
        ],
    )
    def _sc_agg_k(y_hbm, src_hbm, dst_hbm, zeros_hbm, out_hbm,
                  src_v, dst_v, rows0, acc, sem):
        c = lax.axis_index("c")
        s = lax.axis_index("s")
        w = c * NS + s
        pltpu.sync_copy(zeros_hbm.at[pl.ds(s * RPT, RPT)], acc.at[pl.ds(s * RPT, RPT)])
        pltpu.sync_copy(src_hbm.at[w], src_v)
        pltpu.sync_copy(dst_hbm.at[w], dst_v)
        plsc.subcore_barrier()

        def body(j, carry):
            pltpu.async_copy(y_hbm.at[src_v.at[j]], rows0, sem).wait()
            pltpu.sync_copy(rows0, acc.at[dst_v.at[j]], add=True)
            return carry

        lax.fori_loop(0, NCHUNK, body, 0)
        plsc.subcore_barrier()
        pltpu.sync_copy(acc.at[pl.ds(s * RPT, RPT)], out_hbm.at[c, pl.ds(s * RPT, RPT)])

    return _sc_agg_k


def _sc_agg(y, src, dst, zeros_d):
    return _sc_agg_kernel()(y, src, dst, zeros_d)


BLK = 1024
GRID = NPAD // BLK


def _tc_first(feat_ref, w_ref, c_ref, y_ref, dis_ref):
    cnt = jnp.sum(c_ref[...], axis=0)
    dis = lax.rsqrt(cnt + 1.0)[:, None]
    xw = jnp.dot(feat_ref[...], w_ref[...], preferred_element_type=jnp.float32)
    y_ref[...] = xw * dis
    dis_ref[...] = jnp.broadcast_to(dis, (BLK, CW))


def _tc_mid(y_ref, p_ref, dis_ref, b_ref, w_ref, y2_ref):
    d = dis_ref[...][:, 0:1]
    z = jnp.maximum(d * (y_ref[...] + p_ref[0] + p_ref[1]) + b_ref[...], 0.0)
    y2_ref[...] = jnp.dot(z, w_ref[...], preferred_element_type=jnp.float32) * d


def _tc_last(y_ref, p_ref, dis_ref, b_ref, o_ref):
    d = dis_ref[...][:, 0:1]
    o_ref[...] = jnp.maximum(d * (y_ref[...] + p_ref[0] + p_ref[1]) + b_ref[...], 0.0)


def _row_spec(width):
    return pl.BlockSpec((BLK, width), lambda i: (i, 0))


def _pair_spec(width):
    return pl.BlockSpec((NC, BLK, width), lambda i: (0, i, 0))


def _full_spec(shape):
    return pl.BlockSpec(shape, lambda i: tuple(0 for _ in shape))


def kernel(feat, edge_index, W1, b1, W2, b2):
    src = edge_index[0].astype(jnp.int32)
    dst = edge_index[1].astype(jnp.int32)
    pad_e = EPAD - E
    src = jnp.concatenate([src, jnp.full((pad_e,), N, jnp.int32)]).reshape(NW, NCHUNK, CHUNK)
    dst = jnp.concatenate([dst, jnp.full((pad_e,), N, jnp.int32)]).reshape(NW, NCHUNK, CHUNK)
    featp = jnp.zeros((NPAD, D), jnp.float32).at[:N].set(feat)
    zeros_n = jnp.zeros((NPAD,), jnp.float32)
    zeros_d = jnp.zeros((NPAD, D), jnp.float32)
    b1r = b1.reshape(1, D)
    b2r = b2.reshape(1, D)

    cnt = _sc_count(dst, zeros_n)

    y1, dis = pl.pallas_call(
        _tc_first,
        grid=(GRID,),
        in_specs=[_row_spec(D), _full_spec((D, D)),
                  pl.BlockSpec((NW, BLK), lambda i: (0, i))],
        out_specs=[_row_spec(D), _row_spec(CW)],
        out_shape=[jax.ShapeDtypeStruct((NPAD, D), jnp.float32),
                   jax.ShapeDtypeStruct((NPAD, CW), jnp.float32)],
    )(featp, W1, cnt)

    p1 = _sc_agg(y1, src, dst, zeros_d)

    y2 = pl.pallas_call(
        _tc_mid,
        grid=(GRID,),
        in_specs=[_row_spec(D), _pair_spec(D), _row_spec(CW),
                  _full_spec((1, D)), _full_spec((D, D))],
        out_specs=_row_spec(D),
        out_shape=jax.ShapeDtypeStruct((NPAD, D), jnp.float32),
    )(y1, p1, dis, b1r, W2)

    p2 = _sc_agg(y2, src, dst, zeros_d)

    out = pl.pallas_call(
        _tc_last,
        grid=(GRID,),
        in_specs=[_row_spec(D), _pair_spec(D), _row_spec(CW), _full_spec((1, D))],
        out_specs=_row_spec(D),
        out_shape=jax.ShapeDtypeStruct((NPAD, D), jnp.float32),
    )(y2, p2, dis, b2r)

    return out[:N]


# speedup vs baseline: 9.7616x; 1.2624x over previous
import functools

import jax
import jax.numpy as jnp
from jax import lax
from jax.experimental import pallas as pl
from jax.experimental.pallas import tpu as pltpu
from jax.experimental.pallas import tpu_sc as plsc

N = 10000
E = 320000
D = 128

NPAD = 10240
EPAD = 327680
NC, NS = 2, 16
NW = NC * NS
EPT = EPAD // NW
CHUNK = 128
NCHUNK = EPT // CHUNK
RPT = NPAD // NS
CW = 8

def _mesh():
    return plsc.VectorSubcoreMesh(
        core_axis_name="c", subcore_axis_name="s", num_cores=NC, num_subcores=NS)


@functools.cache
def _sc_count_kernel():
    @functools.partial(
        pl.kernel,
        out_type=jax.ShapeDtypeStruct((NW, NPAD), jnp.float32),
        mesh=_mesh(),
        compiler_params=pltpu.CompilerParams(needs_layout_passes=False),
        scratch_types=[
            pltpu.VMEM((NCHUNK, CHUNK), jnp.int32),
            pltpu.VMEM((NPAD,), jnp.float32),
        ],
    )
    def _sc_count(dst_hbm, zeros_hbm, out_hbm, idx_v, cnt_v):
        c = lax.axis_index("c")
        s = lax.axis_index("s")
        w = c * NS + s
        pltpu.sync_copy(zeros_hbm, cnt_v)
        pltpu.sync_copy(dst_hbm.at[pl.ds(w * NCHUNK, NCHUNK)], idx_v)
        ones = jnp.full((16,), 1.0, jnp.float32)

        def body(j, carry):
            for k in range(CHUNK // 16):
                idx = idx_v[j, pl.ds(k * 16, 16)]
                plsc.addupdate_scatter(cnt_v, [idx], ones)
            return carry

        lax.fori_loop(0, NCHUNK, body, 0)
        pltpu.sync_copy(cnt_v, out_hbm.at[w])

    return _sc_count


def _sc_count(dst, zeros_n):
    return _sc_count_kernel()(dst, zeros_n)


NCH_ALL = EPAD // CHUNK
FAST_CORE = 0
QF = 120
QS = NCH_ALL // NS - QF


@functools.cache
def _sc_agg_kernel():
    @functools.partial(
        pl.kernel,
        out_type=jax.ShapeDtypeStruct((NC, NPAD, D), jnp.float32),
        mesh=_mesh(),
        scratch_types=[
            pltpu.VMEM((max(QF, QS), CHUNK), jnp.int32),
            pltpu.VMEM((max(QF, QS), CHUNK), jnp.int32),
            pltpu.VMEM((CHUNK, D), jnp.float32),
            pltpu.VMEM_SHARED((NPAD, D), jnp.float32),
            pltpu.SemaphoreType.DMA,
        ],
    )
    def _sc_agg_k(y_hbm, src_hbm, dst_hbm, zeros_hbm, out_hbm,
                  src_v, dst_v, rows0, acc, sem):
        c = lax.axis_index("c")
        s = lax.axis_index("s")
        nchunk = jnp.where(c == FAST_CORE, QF, QS)
        base = jnp.where(c == FAST_CORE, s * QF, NS * QF + s * QS)
        pltpu.sync_copy(zeros_hbm.at[pl.ds(s * RPT, RPT)], acc.at[pl.ds(s * RPT, RPT)])
        pltpu.sync_copy(src_hbm.at[pl.ds(base, QS)], src_v.at[pl.ds(0, QS)])
        pltpu.sync_copy(dst_hbm.at[pl.ds(base, QS)], dst_v.at[pl.ds(0, QS)])

        @pl.when(c == FAST_CORE)
        def _():
            pltpu.sync_copy(src_hbm.at[pl.ds(base + QS, QF - QS)],
                            src_v.at[pl.ds(QS, QF - QS)])
            pltpu.sync_copy(dst_hbm.at[pl.ds(base + QS, QF - QS)],
                            dst_v.at[pl.ds(QS, QF - QS)])

        plsc.subcore_barrier()

        def body(j, carry):
            pltpu.async_copy(y_hbm.at[src_v.at[j]], rows0, sem).wait()
            pltpu.sync_copy(rows0, acc.at[dst_v.at[j]], add=True)
            return carry

        lax.fori_loop(0, nchunk, body, 0)
        plsc.subcore_barrier()
        pltpu.sync_copy(acc.at[pl.ds(s * RPT, RPT)], out_hbm.at[c, pl.ds(s * RPT, RPT)])

    return _sc_agg_k


def _sc_agg(y, src, dst, zeros_d):
    return _sc_agg_kernel()(y, src, dst, zeros_d)


BLK = 1024
GRID = NPAD // BLK


def _tc_first(feat_ref, w_ref, c_ref, y_ref, dis_ref):
    cnt = jnp.sum(c_ref[...], axis=0)
    dis = lax.rsqrt(cnt + 1.0)[:, None]
    xw = jnp.dot(feat_ref[...], w_ref[...], preferred_element_type=jnp.float32)
    y_ref[...] = xw * dis
    dis_ref[...] = jnp.broadcast_to(dis, (BLK, CW))


def _tc_mid(y_ref, p_ref, dis_ref, b_ref, w_ref, y2_ref):
    d = dis_ref[...][:, 0:1]
    z = jnp.maximum(d * (y_ref[...] + p_ref[0] + p_ref[1]) + b_ref[...], 0.0)
    y2_ref[...] = jnp.dot(z, w_ref[...], preferred_element_type=jnp.float32) * d


def _tc_last(y_ref, p_ref, dis_ref, b_ref, o_ref):
    d = dis_ref[...][:, 0:1]
    o_ref[...] = jnp.maximum(d * (y_ref[...] + p_ref[0] + p_ref[1]) + b_ref[...], 0.0)


def _row_spec(width):
    return pl.BlockSpec((BLK, width), lambda i: (i, 0))


def _pair_spec(width):
    return pl.BlockSpec((NC, BLK, width), lambda i: (0, i, 0))


def _full_spec(shape):
    return pl.BlockSpec(shape, lambda i: tuple(0 for _ in shape))


def kernel(feat, edge_index, W1, b1, W2, b2):
    src = edge_index[0].astype(jnp.int32)
    dst = edge_index[1].astype(jnp.int32)
    pad_e = EPAD - E
    src = jnp.concatenate([src, jnp.full((pad_e,), N, jnp.int32)]).reshape(NCH_ALL, CHUNK)
    dst = jnp.concatenate([dst, jnp.full((pad_e,), N, jnp.int32)]).reshape(NCH_ALL, CHUNK)
    featp = jnp.zeros((NPAD, D), jnp.float32).at[:N].set(feat)
    zeros_n = jnp.zeros((NPAD,), jnp.float32)
    zeros_d = jnp.zeros((NPAD, D), jnp.float32)
    b1r = b1.reshape(1, D)
    b2r = b2.reshape(1, D)

    cnt = _sc_count(dst, zeros_n)

    y1, dis = pl.pallas_call(
        _tc_first,
        grid=(GRID,),
        in_specs=[_row_spec(D), _full_spec((D, D)),
                  pl.BlockSpec((NW, BLK), lambda i: (0, i))],
        out_specs=[_row_spec(D), _row_spec(CW)],
        out_shape=[jax.ShapeDtypeStruct((NPAD, D), jnp.float32),
                   jax.ShapeDtypeStruct((NPAD, CW), jnp.float32)],
    )(featp, W1, cnt)

    p1 = _sc_agg(y1, src, dst, zeros_d)

    y2 = pl.pallas_call(
        _tc_mid,
        grid=(GRID,),
        in_specs=[_row_spec(D), _pair_spec(D), _row_spec(CW),
                  _full_spec((1, D)), _full_spec((D, D))],
        out_specs=_row_spec(D),
        out_shape=jax.ShapeDtypeStruct((NPAD, D), jnp.float32),
    )(y1, p1, dis, b1r, W2)

    p2 = _sc_agg(y2, src, dst, zeros_d)

    out = pl.pallas_call(
        _tc_last,
        grid=(GRID,),
        in_specs=[_row_spec(D), _pair_spec(D), _row_spec(CW), _full_spec((1, D))],
        out_specs=_row_spec(D),
        out_shape=jax.ShapeDtypeStruct((NPAD, D), jnp.float32),
    )(y2, p2, dis, b2r)

    return out[:N]
